# baseline (speedup 1.0000x reference)
import jax
import jax.numpy as jnp
from jax.experimental import pallas as pl
from jax.experimental.pallas import tpu as pltpu
from jax.experimental.pallas import tpu_sc as plsc

_IOU_T = 0.4
_SCORE_T = 0.3
_B = 1024
_NP = 5120
_BC = 512
_NB = _NP // _B

_INTERPRET = False


def _dot01(a, b):
    return jax.lax.dot_general(
        a.astype(jnp.bfloat16), b.astype(jnp.bfloat16),
        (((1,), (0,)), ((), ())),
        preferred_element_type=jnp.float32)


def _iota(shape, dim):
    return jax.lax.broadcasted_iota(jnp.int32, shape, dim)


def _rank_kernel(s_blk_ref, s_row_ref, rank_ref):
    i = pl.program_id(0)
    s_blk = s_blk_ref[...]
    gidx_col = i * _B + _iota((_B, 1), 0)
    acc = jnp.zeros((_B, 1), jnp.float32)
    for m in range(_NB):
        s_chunk = s_row_ref[0:1, m * _B:(m + 1) * _B]
        j_row = m * _B + _iota((1, _B), 1)
        before = (s_chunk > s_blk) | ((s_chunk == s_blk) & (j_row < gidx_col))
        acc = acc + jnp.sum(before.astype(jnp.float32), axis=1, keepdims=True)
    rank_ref[...] = acc.astype(jnp.int32)


def _sc_scatter_body(pay_hbm, rank_hbm, t_hbm, rows_hbm,
                     pay_v, rank_v, out_v):
    wid = jax.lax.axis_index("s") * 2 + jax.lax.axis_index("c")
    lane = jax.lax.iota(jnp.int32, 16)
    ngroups = _NP // 16

    @pl.when(wid == 0)
    def _rows():
        pltpu.sync_copy(pay_hbm, pay_v)
        pltpu.sync_copy(rank_hbm, rank_v)

        def body(g, carry):
            base = g * 16
            idx = rank_v[pl.ds(base, 16)]
            src = (base + lane) * 8
            for c in range(5):
                vals = plsc.load_gather(pay_v, [src + c])
                plsc.store_scatter(out_v, [idx * 8 + c], vals)
            return carry

        jax.lax.fori_loop(0, ngroups, body, 0)
        pltpu.sync_copy(out_v, rows_hbm)

    @pl.when(wid == 1)
    def _t():
        pltpu.sync_copy(pay_hbm, pay_v)
        pltpu.sync_copy(rank_hbm, rank_v)

        def body(g, carry):
            base = g * 16
            idx = rank_v[pl.ds(base, 16)]
            src = (base + lane) * 8
            for c in range(4):
                vals = plsc.load_gather(pay_v, [src + c])
                plsc.store_scatter(out_v, [c * _NP + idx], vals)
            return carry

        jax.lax.fori_loop(0, ngroups, body, 0)
        pltpu.sync_copy(out_v, t_hbm)


def _iou_gt(x1c, y1c, x2c, y2c, ac, x1r, y1r, x2r, y2r, ar):
    ix1 = jnp.maximum(x1c, x1r)
    iy1 = jnp.maximum(y1c, y1r)
    ix2 = jnp.minimum(x2c, x2r)
    iy2 = jnp.minimum(y2c, y2r)
    iw = jnp.clip(ix2 - ix1, 0.0)
    ih = jnp.clip(iy2 - iy1, 0.0)
    inter = iw * ih
    union = ac + ar - inter
    iou = inter / jnp.maximum(union, 1e-9)
    return iou > _IOU_T


def _coords_col(v):
    x1 = v[:, 0:1]
    y1 = v[:, 1:2]
    w = v[:, 2:3]
    h = v[:, 3:4]
    return x1, y1, x1 + w, y1 + h, w * h


def _coords_row(vt, lo, hi):
    x1 = vt[0:1, lo:hi]
    y1 = vt[1:2, lo:hi]
    w = vt[2:3, lo:hi]
    h = vt[3:4, lo:hi]
    return x1, y1, x1 + w, y1 + h, w * h


def _nms_kernel(bt_full_ref, bt_blk_ref, br_blk_ref, v_full_ref,
                keep_ref, boxes_ref, m2_ref):
    k = pl.program_id(0)

    @pl.when(k == 0)
    def _init():
        keep_ref[...] = jnp.zeros_like(keep_ref)
        boxes_ref[...] = jnp.zeros_like(boxes_ref)

    br = br_blk_ref[...]
    x1c, y1c, x2c, y2c, ac = _coords_col(br)
    sc = br[:, 4:5]
    keep0 = (sc > _SCORE_T).astype(jnp.float32)

    def _chunk(m, susp):
        lo = m * _BC
        x1r = bt_full_ref[0:1, pl.ds(lo, _BC)]
        y1r = bt_full_ref[1:2, pl.ds(lo, _BC)]
        wr = bt_full_ref[2:3, pl.ds(lo, _BC)]
        hr = bt_full_ref[3:4, pl.ds(lo, _BC)]
        mk = _iou_gt(x1c, y1c, x2c, y2c, ac,
                     x1r, y1r, x1r + wr, y1r + hr, wr * hr)
        keep_chunk = keep_ref[pl.ds(lo, _BC), 0:1]
        return susp + _dot01(mk, keep_chunk)

    susp = jax.lax.fori_loop(0, k * (_B // _BC), _chunk,
                             jnp.zeros((_B, 1), jnp.float32))

    keep_in = keep0 * (susp < 0.5).astype(jnp.float32)

    bt = bt_blk_ref[...]
    x1r, y1r, x2r, y2r, ar = _coords_row(bt, 0, _B)
    strict = _iota((_B, _B), 1) < _iota((_B, _B), 0)
    m2_ref[...] = (_iou_gt(x1c, y1c, x2c, y2c, ac,
                           x1r, y1r, x2r, y2r, ar)
                   & strict).astype(jnp.bfloat16)

    def _apply(x):
        susp2 = jax.lax.dot_general(
            m2_ref[...], x.astype(jnp.bfloat16), (((1,), (0,)), ((), ())),
            preferred_element_type=jnp.float32)
        return keep_in * (susp2 < 0.5).astype(jnp.float32)

    def _cond(c):
        x, fx = c
        return jnp.max(jnp.abs(x - fx)) > 0.0

    def _body(c):
        _, fx = c
        a = _apply(fx)
        return a, _apply(a)

    x0 = keep_in
    _, keep_fin = jax.lax.while_loop(_cond, _body, (x0, _apply(x0)))

    keep_ref[pl.ds(k * _B, _B), :] = keep_fin

    @pl.when(k == _NB - 1)
    def _final():
        boxes_ref[...] = v_full_ref[:, 0:4] * keep_ref[...]


def kernel(boxes, scores):
    n = boxes.shape[0]
    pad = _NP - n
    b = jnp.pad(boxes, ((0, pad), (0, 0)))
    s = jnp.pad(scores, (0, pad), constant_values=-1.0)
    v = jnp.concatenate(
        [b, s[:, None], jnp.zeros((_NP, 3), jnp.float32)], axis=1)
    s_col = s[:, None]
    s_row = s[None, :]

    rank = pl.pallas_call(
        _rank_kernel,
        grid=(_NB,),
        in_specs=[
            pl.BlockSpec((_B, 1), lambda i: (i, 0)),
            pl.BlockSpec((1, _NP), lambda i: (0, 0)),
        ],
        out_specs=pl.BlockSpec((_B, 1), lambda i: (i, 0)),
        out_shape=jax.ShapeDtypeStruct((_NP, 1), jnp.int32),
        interpret=_INTERPRET,
    )(s_col, s_row)

    sc_scatter = pl.kernel(
        _sc_scatter_body,
        out_type=[
            jax.ShapeDtypeStruct((8 * _NP,), jnp.float32),
            jax.ShapeDtypeStruct((_NP * 8,), jnp.float32),
        ],
        mesh=plsc.VectorSubcoreMesh(core_axis_name="c", subcore_axis_name="s",
                                    num_cores=2, num_subcores=16),
        scratch_types=[
            pltpu.VMEM((_NP * 8,), jnp.float32),
            pltpu.VMEM((_NP,), jnp.int32),
            pltpu.VMEM((_NP * 8,), jnp.float32),
        ],
        compiler_params=pltpu.CompilerParams(needs_layout_passes=False),
        interpret=_INTERPRET,
    )
    t_flat, rows_flat = sc_scatter(v.reshape(-1), rank.reshape(-1))
    st = t_flat.reshape(8, _NP)
    sr = rows_flat.reshape(_NP, 8)

    keepf, ob = pl.pallas_call(
        _nms_kernel,
        grid=(_NB,),
        in_specs=[
            pl.BlockSpec((8, _NP), lambda k: (0, 0)),
            pl.BlockSpec((8, _B), lambda k: (0, k)),
            pl.BlockSpec((_B, 8), lambda k: (k, 0)),
            pl.BlockSpec((_NP, 8), lambda k: (0, 0)),
        ],
        out_specs=[
            pl.BlockSpec((_NP, 1), lambda k: (0, 0)),
            pl.BlockSpec((_NP, 4), lambda k: (0, 0)),
        ],
        out_shape=[
            jax.ShapeDtypeStruct((_NP, 1), jnp.float32),
            jax.ShapeDtypeStruct((_NP, 4), jnp.float32),
        ],
        scratch_shapes=[
            pltpu.VMEM((_B, _B), jnp.bfloat16),
        ],
        interpret=_INTERPRET,
    )(st, st, sr, sr)

    out_boxes = ob[:n]
    keep = keepf[:n, 0].astype(jnp.bool_)
    return out_boxes, keep


# speedup vs baseline: 1.0819x; 1.0819x over previous
import jax
import jax.numpy as jnp
from jax.experimental import pallas as pl
from jax.experimental.pallas import tpu as pltpu
from jax.experimental.pallas import tpu_sc as plsc

_IOU_T = 0.4
_SCORE_T = 0.3
_B = 1024
_NP = 5120
_BC = 1024
_NB = _NP // _B

_INTERPRET = False


def _dot01(a, b):
    return jax.lax.dot_general(
        a.astype(jnp.bfloat16), b.astype(jnp.bfloat16),
        (((1,), (0,)), ((), ())),
        preferred_element_type=jnp.float32)


def _iota(shape, dim):
    return jax.lax.broadcasted_iota(jnp.int32, shape, dim)


def _rank_kernel(s_blk_ref, s_row_ref, rank_ref):
    i = pl.program_id(0)
    s_blk = s_blk_ref[...]
    gidx_col = i * _B + _iota((_B, 1), 0)
    acc = jnp.zeros((_B, 1), jnp.float32)
    for m in range(_NB):
        s_chunk = s_row_ref[0:1, m * _B:(m + 1) * _B]
        j_row = m * _B + _iota((1, _B), 1)
        before = (s_chunk > s_blk) | ((s_chunk == s_blk) & (j_row < gidx_col))
        acc = acc + jnp.sum(before.astype(jnp.float32), axis=1, keepdims=True)
    rank_ref[...] = acc.astype(jnp.int32)


def _sc_scatter_body(pay_hbm, rank_hbm, t_hbm, rows_hbm,
                     pay_v, rank_v, out_v):
    wid = jax.lax.axis_index("s") * 2 + jax.lax.axis_index("c")
    lane = jax.lax.iota(jnp.int32, 16)
    ngroups = _NP // 16

    @pl.when(wid == 0)
    def _rows():
        pltpu.sync_copy(pay_hbm, pay_v)
        pltpu.sync_copy(rank_hbm, rank_v)

        def body(g, carry):
            base = g * 16
            idx = rank_v[pl.ds(base, 16)]
            src = (base + lane) * 8
            for c in range(5):
                vals = plsc.load_gather(pay_v, [src + c])
                plsc.store_scatter(out_v, [idx * 8 + c], vals)
            return carry

        jax.lax.fori_loop(0, ngroups, body, 0)
        pltpu.sync_copy(out_v, rows_hbm)

    @pl.when(wid == 1)
    def _t():
        pltpu.sync_copy(pay_hbm, pay_v)
        pltpu.sync_copy(rank_hbm, rank_v)

        def body(g, carry):
            base = g * 16
            idx = rank_v[pl.ds(base, 16)]
            src = (base + lane) * 8
            for c in range(4):
                vals = plsc.load_gather(pay_v, [src + c])
                plsc.store_scatter(out_v, [c * _NP + idx], vals)
            return carry

        jax.lax.fori_loop(0, ngroups, body, 0)
        pltpu.sync_copy(out_v, t_hbm)


def _iou_gt(x1c, y1c, x2c, y2c, ac, x1r, y1r, x2r, y2r, ar):
    ix1 = jnp.maximum(x1c, x1r)
    iy1 = jnp.maximum(y1c, y1r)
    ix2 = jnp.minimum(x2c, x2r)
    iy2 = jnp.minimum(y2c, y2r)
    iw = jnp.clip(ix2 - ix1, 0.0)
    ih = jnp.clip(iy2 - iy1, 0.0)
    inter = iw * ih
    union = ac + ar - inter
    iou = inter / jnp.maximum(union, 1e-9)
    return iou > _IOU_T


def _coords_col(v):
    x1 = v[:, 0:1]
    y1 = v[:, 1:2]
    w = v[:, 2:3]
    h = v[:, 3:4]
    return x1, y1, x1 + w, y1 + h, w * h


def _coords_row(vt, lo, hi):
    x1 = vt[0:1, lo:hi]
    y1 = vt[1:2, lo:hi]
    w = vt[2:3, lo:hi]
    h = vt[3:4, lo:hi]
    return x1, y1, x1 + w, y1 + h, w * h


def _nms_kernel(bt_full_ref, bt_blk_ref, br_blk_ref, v_full_ref,
                keep_ref, boxes_ref, m2_ref):
    k = pl.program_id(0)

    @pl.when(k == 0)
    def _init():
        keep_ref[...] = jnp.zeros_like(keep_ref)
        boxes_ref[...] = jnp.zeros_like(boxes_ref)

    br = br_blk_ref[...]
    x1c, y1c, x2c, y2c, ac = _coords_col(br)
    sc = br[:, 4:5]
    keep0 = (sc > _SCORE_T).astype(jnp.float32)

    def _chunk(m, susp):
        lo = m * _BC
        x1r = bt_full_ref[0:1, pl.ds(lo, _BC)]
        y1r = bt_full_ref[1:2, pl.ds(lo, _BC)]
        wr = bt_full_ref[2:3, pl.ds(lo, _BC)]
        hr = bt_full_ref[3:4, pl.ds(lo, _BC)]
        mk = _iou_gt(x1c, y1c, x2c, y2c, ac,
                     x1r, y1r, x1r + wr, y1r + hr, wr * hr)
        keep_chunk = keep_ref[pl.ds(lo, _BC), 0:1]
        return susp + _dot01(mk, keep_chunk)

    susp = jax.lax.fori_loop(0, k * (_B // _BC), _chunk,
                             jnp.zeros((_B, 1), jnp.float32))

    keep_in = keep0 * (susp < 0.5).astype(jnp.float32)

    bt = bt_blk_ref[...]
    x1r, y1r, x2r, y2r, ar = _coords_row(bt, 0, _B)
    strict = _iota((_B, _B), 1) < _iota((_B, _B), 0)
    m2_ref[...] = (_iou_gt(x1c, y1c, x2c, y2c, ac,
                           x1r, y1r, x2r, y2r, ar)
                   & strict).astype(jnp.bfloat16)

    def _apply(x):
        susp2 = jax.lax.dot_general(
            m2_ref[...], x.astype(jnp.bfloat16), (((1,), (0,)), ((), ())),
            preferred_element_type=jnp.float32)
        return keep_in * (susp2 < 0.5).astype(jnp.float32)

    def _cond(c):
        x, fx = c
        return jnp.max(jnp.abs(x - fx)) > 0.0

    def _body(c):
        _, fx = c
        a = _apply(fx)
        return a, _apply(a)

    x0 = keep_in
    _, keep_fin = jax.lax.while_loop(_cond, _body, (x0, _apply(x0)))

    keep_ref[pl.ds(k * _B, _B), :] = keep_fin

    @pl.when(k == _NB - 1)
    def _final():
        boxes_ref[...] = v_full_ref[:, 0:4] * keep_ref[...]


def kernel(boxes, scores):
    n = boxes.shape[0]
    pad = _NP - n
    b = jnp.pad(boxes, ((0, pad), (0, 0)))
    s = jnp.pad(scores, (0, pad), constant_values=-1.0)
    v = jnp.concatenate(
        [b, s[:, None], jnp.zeros((_NP, 3), jnp.float32)], axis=1)
    s_col = s[:, None]
    s_row = s[None, :]

    rank = pl.pallas_call(
        _rank_kernel,
        grid=(_NB,),
        in_specs=[
            pl.BlockSpec((_B, 1), lambda i: (i, 0)),
            pl.BlockSpec((1, _NP), lambda i: (0, 0)),
        ],
        out_specs=pl.BlockSpec((_B, 1), lambda i: (i, 0)),
        out_shape=jax.ShapeDtypeStruct((_NP, 1), jnp.int32),
        interpret=_INTERPRET,
    )(s_col, s_row)

    sc_scatter = pl.kernel(
        _sc_scatter_body,
        out_type=[
            jax.ShapeDtypeStruct((8 * _NP,), jnp.float32),
            jax.ShapeDtypeStruct((_NP * 8,), jnp.float32),
        ],
        mesh=plsc.VectorSubcoreMesh(core_axis_name="c", subcore_axis_name="s",
                                    num_cores=2, num_subcores=16),
        scratch_types=[
            pltpu.VMEM((_NP * 8,), jnp.float32),
            pltpu.VMEM((_NP,), jnp.int32),
            pltpu.VMEM((_NP * 8,), jnp.float32),
        ],
        compiler_params=pltpu.CompilerParams(needs_layout_passes=False),
        interpret=_INTERPRET,
    )
    t_flat, rows_flat = sc_scatter(v.reshape(-1), rank.reshape(-1))
    st = t_flat.reshape(8, _NP)
    sr = rows_flat.reshape(_NP, 8)

    keepf, ob = pl.pallas_call(
        _nms_kernel,
        grid=(_NB,),
        in_specs=[
            pl.BlockSpec((8, _NP), lambda k: (0, 0)),
            pl.BlockSpec((8, _B), lambda k: (0, k)),
            pl.BlockSpec((_B, 8), lambda k: (k, 0)),
            pl.BlockSpec((_NP, 8), lambda k: (0, 0)),
        ],
        out_specs=[
            pl.BlockSpec((_NP, 1), lambda k: (0, 0)),
            pl.BlockSpec((_NP, 4), lambda k: (0, 0)),
        ],
        out_shape=[
            jax.ShapeDtypeStruct((_NP, 1), jnp.float32),
            jax.ShapeDtypeStruct((_NP, 4), jnp.float32),
        ],
        scratch_shapes=[
            pltpu.VMEM((_B, _B), jnp.bfloat16),
        ],
        interpret=_INTERPRET,
    )(st, st, sr, sr)

    out_boxes = ob[:n]
    keep = keepf[:n, 0].astype(jnp.bool_)
    return out_boxes, keep
